# Initial kernel scaffold; baseline (speedup 1.0000x reference)
import jax
import jax.numpy as jnp
from jax.experimental import pallas as pl


def kernel(x, edge_index, batch, W_in, b_in, W0, b0, W1, b1, W2, b2, g0, be0, g1, be1, g2, be2, W_out, b_out):
    raise NotImplementedError("write your pallas kernel here")


# speedup vs baseline: 8.5882x; 8.5882x over previous
import functools

import jax
import jax.numpy as jnp
from jax import lax
from jax.experimental import pallas as pl
from jax.experimental.pallas import tpu as pltpu
from jax.experimental.pallas import tpu_sc as plsc

NC = 2
NS = 16
NW = NC * NS
C = 128
R = 1000
DW = 16


def _sc_mesh():
    return plsc.VectorSubcoreMesh(
        core_axis_name="c", subcore_axis_name="s",
        num_cores=NC, num_subcores=NS)


def _sc_scatter_build(n, d, e_pad, n_acc):
    ew = e_pad // NW
    steps = ew // C
    zr = n_acc // NS

    @functools.partial(
        pl.kernel,
        out_type=jax.ShapeDtypeStruct((NC, n_acc, d), jnp.float32),
        mesh=_sc_mesh(),
        scratch_types=[
            pltpu.VMEM((C,), jnp.int32),
            pltpu.VMEM((C,), jnp.int32),
            pltpu.VMEM((C, d), jnp.float32),
            pltpu.VMEM_SHARED((n_acc, d), jnp.float32),
            pltpu.SemaphoreType.DMA,
        ],
    )
    def sc(y_hbm, src_hbm, dst_hbm, zeros_hbm, out_hbm,
           src_v, dst_v, rows_v, acc, sem):
        cid = lax.axis_index("c")
        sid = lax.axis_index("s")
        wid = sid * NC + cid
        pltpu.sync_copy(zeros_hbm, acc.at[pl.ds(sid * zr, zr)])
        plsc.subcore_barrier()
        base = wid * ew

        def body(i, carry):
            off = base + i * C
            pltpu.sync_copy(src_hbm.at[pl.ds(off, C)], src_v)
            pltpu.sync_copy(dst_hbm.at[pl.ds(off, C)], dst_v)
            pltpu.async_copy(y_hbm.at[src_v], rows_v, sem).wait()
            pltpu.sync_copy(rows_v, acc.at[dst_v], add=True)
            return carry

        lax.fori_loop(0, steps, body, 0)
        plsc.subcore_barrier()
        pltpu.sync_copy(acc.at[pl.ds(sid * zr, zr)],
                        out_hbm.at[cid, pl.ds(sid * zr, zr)])

    return sc


def _sc_deg_build(n, e_pad, n_acc):
    ew = e_pad // NW
    steps = ew // C
    zr = n_acc // NS

    @functools.partial(
        pl.kernel,
        out_type=jax.ShapeDtypeStruct((NC, n_acc, DW), jnp.float32),
        mesh=_sc_mesh(),
        scratch_types=[
            pltpu.VMEM((C,), jnp.int32),
            pltpu.VMEM((C, DW), jnp.float32),
            pltpu.VMEM_SHARED((n_acc, DW), jnp.float32),
        ],
    )
    def sc(dst_hbm, zeros_hbm, ones_hbm, out_hbm, dst_v, ones_v, acc):
        cid = lax.axis_index("c")
        sid = lax.axis_index("s")
        wid = sid * NC + cid
        pltpu.sync_copy(zeros_hbm, acc.at[pl.ds(sid * zr, zr)])
        pltpu.sync_copy(ones_hbm, ones_v)
        plsc.subcore_barrier()
        base = wid * ew

        def body(i, carry):
            off = base + i * C
            pltpu.sync_copy(dst_hbm.at[pl.ds(off, C)], dst_v)
            pltpu.sync_copy(ones_v, acc.at[dst_v], add=True)
            return carry

        lax.fori_loop(0, steps, body, 0)
        plsc.subcore_barrier()
        pltpu.sync_copy(acc.at[pl.ds(sid * zr, zr)],
                        out_hbm.at[cid, pl.ds(sid * zr, zr)])

    return sc


def _tc_pre_build(n, d):
    nb = n // R

    def body(x_b, wi_b, bi_b, w0_b, h_b, xw_b):
        h = jnp.dot(x_b[...], wi_b[...],
                    preferred_element_type=jnp.float32) + bi_b[...]
        h_b[...] = h
        xw_b[...] = jnp.dot(h, w0_b[...], preferred_element_type=jnp.float32)

    full = pl.BlockSpec((d, d), lambda i: (0, 0))
    bias = pl.BlockSpec((1, d), lambda i: (0, 0))
    rows = pl.BlockSpec((R, d), lambda i: (i, 0))
    return pl.pallas_call(
        body,
        grid=(nb,),
        in_specs=[rows, full, bias, full],
        out_specs=[rows, rows],
        out_shape=[jax.ShapeDtypeStruct((n, d), jnp.float32),
                   jax.ShapeDtypeStruct((n, d), jnp.float32)],
    )


def _tc_dinv_build(n, d):
    nb = n // R

    def body(d0_b, d1_b, xw_b, dinv_b, y_b):
        cnt = d0_b[0, :, 0:1] + d1_b[0, :, 0:1]
        dinv = lax.rsqrt(cnt + 1.0)
        dinv_b[...] = dinv
        y_b[...] = xw_b[...] * dinv

    degs = pl.BlockSpec((1, R, DW), lambda i: (0, i, 0))
    degs2 = pl.BlockSpec((1, R, DW), lambda i: (1, i, 0))
    rows = pl.BlockSpec((R, d), lambda i: (i, 0))
    col = pl.BlockSpec((R, 1), lambda i: (i, 0))
    return pl.pallas_call(
        body,
        grid=(nb,),
        in_specs=[degs, degs2, rows],
        out_specs=[col, rows],
        out_shape=[jax.ShapeDtypeStruct((n, 1), jnp.float32),
                   jax.ShapeDtypeStruct((n, d), jnp.float32)],
    )


def _tc_mid_build(n, d, with_next):
    nb = n // R

    def body(z0_b, z1_b, y_b, prev_b, dinv_b, b_b, g_b, be_b, *rest):
        if with_next:
            wn_b, h_b, yn_b = rest
        else:
            (h_b,) = rest
        dinv = dinv_b[...]
        t = (z0_b[0] + z1_b[0] + y_b[...]) * dinv + b_b[...]
        t = jnp.maximum(t, 0.0)
        mu = jnp.mean(t, axis=1, keepdims=True)
        var = jnp.mean((t - mu) ** 2, axis=1, keepdims=True)
        t = (t - mu) * lax.rsqrt(var + 1e-5) * g_b[...] + be_b[...]
        h = t + prev_b[...]
        h_b[...] = h
        if with_next:
            yn_b[...] = jnp.dot(h, wn_b[...],
                                preferred_element_type=jnp.float32) * dinv

    rows = pl.BlockSpec((R, d), lambda i: (i, 0))
    part0 = pl.BlockSpec((1, R, d), lambda i: (0, i, 0))
    part1 = pl.BlockSpec((1, R, d), lambda i: (1, i, 0))
    col = pl.BlockSpec((R, 1), lambda i: (i, 0))
    bias = pl.BlockSpec((1, d), lambda i: (0, 0))
    full = pl.BlockSpec((d, d), lambda i: (0, 0))
    in_specs = [part0, part1, rows, rows, col, bias, bias, bias]
    out_specs = [rows]
    out_shape = [jax.ShapeDtypeStruct((n, d), jnp.float32)]
    if with_next:
        in_specs.append(full)
        out_specs.append(rows)
        out_shape.append(jax.ShapeDtypeStruct((n, d), jnp.float32))
    return pl.pallas_call(
        body, grid=(nb,), in_specs=in_specs,
        out_specs=out_specs, out_shape=out_shape)


def _tc_pool_build(n, d, g_groups):
    nb = n // R

    def body(emb_b, batch_b, wo_b, bo_b, out_b, sums, cnts):
        i = pl.program_id(0)
        iota = lax.broadcasted_iota(jnp.int32, (R, g_groups), 1)
        oh = (batch_b[...] == iota).astype(jnp.float32)
        dn = (((0,), (0,)), ((), ()))
        bsum = lax.dot_general(oh, emb_b[...], dn,
                               preferred_element_type=jnp.float32)
        bcnt = lax.dot_general(oh, jnp.ones((R, d), jnp.float32), dn,
                               preferred_element_type=jnp.float32)

        @pl.when(i == 0)
        def _():
            sums[...] = bsum
            cnts[...] = bcnt

        @pl.when(i > 0)
        def _():
            sums[...] = sums[...] + bsum
            cnts[...] = cnts[...] + bcnt

        @pl.when(i == nb - 1)
        def _():
            mean = sums[...] / jnp.maximum(cnts[...], 1.0)
            out_b[...] = jnp.dot(mean, wo_b[...],
                                 preferred_element_type=jnp.float32) + bo_b[...]

    rows = pl.BlockSpec((R, d), lambda i: (i, 0))
    col = pl.BlockSpec((R, 1), lambda i: (i, 0))
    full = pl.BlockSpec((d, d), lambda i: (0, 0))
    bias = pl.BlockSpec((1, d), lambda i: (0, 0))
    gout = pl.BlockSpec((g_groups, d), lambda i: (0, 0))
    return pl.pallas_call(
        body,
        grid=(nb,),
        in_specs=[rows, col, full, bias],
        out_specs=gout,
        out_shape=jax.ShapeDtypeStruct((g_groups, d), jnp.float32),
        scratch_shapes=[pltpu.VMEM((g_groups, d), jnp.float32),
                        pltpu.VMEM((g_groups, d), jnp.float32)],
    )


def kernel(x, edge_index, batch, W_in, b_in, W0, b0, W1, b1, W2, b2,
           g0, be0, g1, be1, g2, be2, W_out, b_out):
    n, d = x.shape
    e = edge_index.shape[1]
    g_groups = 16

    e_pad = -(-e // (NW * C)) * (NW * C)
    n_acc = -(-(n + 1) // (NS * 8)) * (NS * 8)
    zr = n_acc // NS

    src = edge_index[0]
    dst = edge_index[1]
    srcp = jnp.concatenate(
        [src, jnp.zeros((e_pad - e,), jnp.int32)]) if e_pad > e else src
    dstp = jnp.concatenate(
        [dst, jnp.full((e_pad - e,), n, jnp.int32)]) if e_pad > e else dst

    zeros_w = jnp.zeros((zr, d), jnp.float32)
    zeros_d = jnp.zeros((zr, DW), jnp.float32)
    ones_d = jnp.ones((C, DW), jnp.float32)

    sc_deg = _sc_deg_build(n, e_pad, n_acc)
    sc_scatter = _sc_scatter_build(n, d, e_pad, n_acc)
    tc_pre = _tc_pre_build(n, d)
    tc_dinv = _tc_dinv_build(n, d)
    tc_mid = _tc_mid_build(n, d, True)
    tc_last = _tc_mid_build(n, d, False)
    tc_pool = _tc_pool_build(n, d, g_groups)

    b2d = lambda v: v.reshape(1, d)

    degp = sc_deg(dstp, zeros_d, ones_d)
    h0, xw1 = tc_pre(x, W_in, b2d(b_in), W0)
    dinv, y = tc_dinv(degp, degp, xw1)

    prev = h0
    zp = sc_scatter(y, srcp, dstp, zeros_w)
    h, y = tc_mid(zp, zp, y, prev, dinv, b2d(b0), b2d(g0), b2d(be0), W1)
    prev = h
    zp = sc_scatter(y, srcp, dstp, zeros_w)
    h, y = tc_mid(zp, zp, y, prev, dinv, b2d(b1), b2d(g1), b2d(be1), W2)
    prev = h
    zp = sc_scatter(y, srcp, dstp, zeros_w)
    (h,) = tc_last(zp, zp, y, prev, dinv, b2d(b2), b2d(g2), b2d(be2))

    graph = tc_pool(h, batch.reshape(n, 1), W_out, b2d(b_out))
    return graph, h
